# baseline (speedup 1.0000x reference)
import jax
import jax.numpy as jnp
from jax import lax
from jax.experimental import pallas as pl
from jax.experimental.pallas import tpu as pltpu
from jax.experimental.pallas import tpu_sc as plsc

NUM_EMB = 1000000
DIM = 32
BATCH = 16384
HIST = 50
HDIM = DIM // 2

_info = plsc.get_sparse_core_info()
NC, NS = _info.num_cores, _info.num_subcores
BW = 512
NPASS = BATCH // (NS * BW)
CH = 1000
NCHT = 62


def _body(tt_hbm, xt_hbm, out_hbm, half_hbm, idx_all,
          rows0, rows1, tb0, tb1, a0, a1, b0, b1,
          gsem0, gsem1, osem0, osem1, asem0, asem1, bsem0, bsem1):
    c = lax.axis_index("c")
    s = lax.axis_index("s")
    lanes = lax.iota(jnp.int32, 16)
    half_c = half_hbm.at[c]

    def fire_a(m, a, sem):
        pltpu.async_copy(
            tt_hbm.at[pl.ds(c * HDIM, HDIM), pl.ds(m * CH, CH)],
            a.at[:, pl.ds(0, CH)], sem)

    def drain_a(a, sem):
        pltpu.make_async_copy(
            tt_hbm.at[pl.ds(0, HDIM), pl.ds(0, CH)],
            a.at[:, pl.ds(0, CH)], sem).wait()

    def transpose_a(a, b):
        @plsc.parallel_loop(0, CH, unroll=4)
        def tra(j):
            jsplat = jnp.full((16,), 0, jnp.int32) + j
            b[j, pl.ds(0, HDIM)] = plsc.load_gather(a, [lanes, jsplat])

    def fire_b(m, b, sem):
        pltpu.async_copy(b, half_c.at[pl.ds(m * CH, CH)], sem)

    def wait_b(b, sem):
        pltpu.make_async_copy(b, half_c.at[pl.ds(0, CH)], sem).wait()

    fire_a(s, a0, asem0)

    def stepa(i, _):
        ma = s + 32 * i
        fire_a(ma + 16, a1, asem1)
        drain_a(a0, asem0)

        @pl.when(i > 0)
        def _():
            wait_b(b0, bsem0)
        transpose_a(a0, b0)
        fire_b(ma, b0, bsem0)

        @pl.when(i < NCHT // 2 - 1)
        def _():
            fire_a(ma + 32, a0, asem0)
        drain_a(a1, asem1)

        @pl.when(i > 0)
        def _():
            wait_b(b1, bsem1)
        transpose_a(a1, b1)
        fire_b(ma + 16, b1, bsem1)
        return ()

    lax.fori_loop(0, NCHT // 2, stepa, ())

    @pl.when(s < 8)
    def _():
        m = 992 + s
        fire_a(m, a0, asem0)
        drain_a(a0, asem0)
        wait_b(b0, bsem0)
        transpose_a(a0, b0)
        fire_b(m, b0, bsem0)

    wait_b(b0, bsem0)
    wait_b(b1, bsem1)
    plsc.subcore_barrier()

    def phase_b(bbase):
        pltpu.sync_copy(xt_hbm.at[:, pl.ds(bbase, BW)], idx_all)

        def fire(l, rows, sem):
            pltpu.async_copy(half_c.at[idx_all.at[l]], rows, sem)

        def drain(rows, sem):
            pltpu.make_async_copy(
                half_c.at[idx_all.at[0]], rows, sem).wait()

        def transpose(rows, tb):
            @plsc.parallel_loop(0, BW, unroll=8)
            def tr(j):
                jsplat = jnp.full((16,), 0, jnp.int32) + j
                plsc.store_scatter(
                    tb, [lanes, jsplat], rows[j, pl.ds(0, HDIM)])

        def out_slice(l):
            return out_hbm.at[l].at[pl.ds(c * HDIM, HDIM), pl.ds(bbase, BW)]

        def wait_out(tb, sem):
            pltpu.make_async_copy(
                tb.at[:, pl.ds(0, BW)], out_slice(0), sem).wait()

        fire(0, rows0, gsem0)

        def step(i, _):
            l = 2 * i

            @pl.when(i > 0)
            def _():
                wait_out(tb0, osem0)
            fire(l + 1, rows1, gsem1)
            drain(rows0, gsem0)
            transpose(rows0, tb0)
            pltpu.async_copy(tb0.at[:, pl.ds(0, BW)], out_slice(l), osem0)

            @pl.when(i > 0)
            def _():
                wait_out(tb1, osem1)
            @pl.when(i < HIST // 2 - 1)
            def _():
                fire(l + 2, rows0, gsem0)
            drain(rows1, gsem1)
            transpose(rows1, tb1)
            pltpu.async_copy(tb1.at[:, pl.ds(0, BW)], out_slice(l + 1), osem1)
            return ()

        lax.fori_loop(0, HIST // 2, step, ())
        wait_out(tb0, osem0)
        wait_out(tb1, osem1)

    for p in range(NPASS):
        phase_b(p * (NS * BW) + s * BW)


@jax.jit
def _gather_t(tt, xt):
    mesh = plsc.VectorSubcoreMesh(core_axis_name="c", subcore_axis_name="s")
    return pl.kernel(
        _body,
        out_type=(
            jax.ShapeDtypeStruct((HIST, DIM, BATCH), jnp.float32),
            jax.ShapeDtypeStruct((NC, NUM_EMB, HDIM), jnp.float32),
        ),
        mesh=mesh,
        scratch_types=[
            pltpu.VMEM((HIST, BW), jnp.int32),
            pltpu.VMEM((BW, HDIM), jnp.float32),
            pltpu.VMEM((BW, HDIM), jnp.float32),
            pltpu.VMEM((HDIM, BW + 1), jnp.float32),
            pltpu.VMEM((HDIM, BW + 1), jnp.float32),
            pltpu.VMEM((HDIM, CH + 7), jnp.float32),
            pltpu.VMEM((HDIM, CH + 7), jnp.float32),
            pltpu.VMEM((CH, HDIM), jnp.float32),
            pltpu.VMEM((CH, HDIM), jnp.float32),
            pltpu.SemaphoreType.DMA,
            pltpu.SemaphoreType.DMA,
            pltpu.SemaphoreType.DMA,
            pltpu.SemaphoreType.DMA,
            pltpu.SemaphoreType.DMA,
            pltpu.SemaphoreType.DMA,
            pltpu.SemaphoreType.DMA,
            pltpu.SemaphoreType.DMA,
        ],
        compiler_params=pltpu.CompilerParams(
            use_tc_tiling_on_sc=False, needs_layout_passes=False),
    )(tt, xt)


def kernel(x, table):
    xt = x.T.astype(jnp.int32)
    tt = table.T
    out_t, _ = _gather_t(tt, xt)
    return jnp.transpose(out_t, (2, 0, 1))


# speedup vs baseline: 4.1719x; 4.1719x over previous
import jax
import jax.numpy as jnp
from jax import lax
from jax.experimental import pallas as pl
from jax.experimental.pallas import tpu as pltpu
from jax.experimental.pallas import tpu_sc as plsc

NUM_EMB = 1000000
DIM = 32
BATCH = 16384
HIST = 50

_info = plsc.get_sparse_core_info()
NC, NS = _info.num_cores, _info.num_subcores
NW = NC * NS
B_PER_W = BATCH // NW
G = 128
NGPL = B_PER_W // G


def _body(table_hbm, xt_hbm, out_hbm, idx_all,
          rows0, rows1, tb0, tb1, gsem0, gsem1, osem0, osem1):
    wid = lax.axis_index("s") * NC + lax.axis_index("c")
    bbase = wid * B_PER_W
    lanes = lax.iota(jnp.int32, 16)
    cols = [jnp.full((16,), dd, jnp.int32) for dd in range(DIM)]

    pltpu.sync_copy(xt_hbm.at[:, pl.ds(bbase, B_PER_W)], idx_all)

    def fire(l, rows, sem):
        pltpu.async_copy(table_hbm.at[idx_all.at[l]], rows, sem)

    def drain(rows, sem):
        pltpu.make_async_copy(
            table_hbm.at[idx_all.at[0]], rows, sem).wait()

    def transpose(rows, tb):
        @plsc.parallel_loop(0, B_PER_W, unroll=8)
        def tr(j):
            jsplat = jnp.full((16,), 0, jnp.int32) + j
            lo = rows[j, pl.ds(0, 16)]
            hi = rows[j, pl.ds(16, 16)]
            plsc.store_scatter(tb, [lanes, jsplat], lo)
            plsc.store_scatter(tb, [lanes + 16, jsplat], hi)

    def out_slice(l):
        return out_hbm.at[l].at[:, pl.ds(bbase, B_PER_W)]

    def wait_out(tb, sem):
        pltpu.make_async_copy(
            tb.at[:, pl.ds(0, B_PER_W)], out_slice(0), sem).wait()

    fire(0, rows0, gsem0)

    def step(i, _):
        l = 2 * i

        @pl.when(i > 0)
        def _():
            wait_out(tb0, osem0)
        fire(l + 1, rows1, gsem1)
        drain(rows0, gsem0)
        transpose(rows0, tb0)
        pltpu.async_copy(tb0.at[:, pl.ds(0, B_PER_W)], out_slice(l), osem0)

        @pl.when(i > 0)
        def _():
            wait_out(tb1, osem1)
        @pl.when(i < HIST // 2 - 1)
        def _():
            fire(l + 2, rows0, gsem0)
        drain(rows1, gsem1)
        transpose(rows1, tb1)
        pltpu.async_copy(tb1.at[:, pl.ds(0, B_PER_W)], out_slice(l + 1), osem1)
        return ()

    lax.fori_loop(0, HIST // 2, step, ())
    wait_out(tb0, osem0)
    wait_out(tb1, osem1)


@jax.jit
def _gather_t(table, xt):
    mesh = plsc.VectorSubcoreMesh(core_axis_name="c", subcore_axis_name="s")
    return pl.kernel(
        _body,
        out_type=jax.ShapeDtypeStruct((HIST, DIM, BATCH), jnp.float32),
        mesh=mesh,
        scratch_types=[
            pltpu.VMEM((HIST, B_PER_W), jnp.int32),
            pltpu.VMEM((B_PER_W, DIM), jnp.float32),
            pltpu.VMEM((B_PER_W, DIM), jnp.float32),
            pltpu.VMEM((DIM, B_PER_W + 1), jnp.float32),
            pltpu.VMEM((DIM, B_PER_W + 1), jnp.float32),
            pltpu.SemaphoreType.DMA,
            pltpu.SemaphoreType.DMA,
            pltpu.SemaphoreType.DMA,
            pltpu.SemaphoreType.DMA,
        ],
        compiler_params=pltpu.CompilerParams(
            use_tc_tiling_on_sc=False, needs_layout_passes=False),
    )(table, xt)


def kernel(x, table):
    xt = x.T.astype(jnp.int32)
    out_t = _gather_t(table, xt)
    return jnp.transpose(out_t, (2, 0, 1))
